# baseline (speedup 1.0000x reference)
import jax
import jax.numpy as jnp
from jax.experimental import pallas as pl
from jax.experimental.pallas import tpu as pltpu

_N = 4096
_HID = 256
_NH = 8
_DH = 32
_BLK = 512
_NBLK = _N // _BLK
_QR = _NH * 8


def _fused(*refs):
    (x_ref,
     xo1, xo2, xo3, xo4, xo5, xo6,
     s0W1, s0b1, s0W2, s0b2, s1W1, s1b1, s1W2, s1b2,
     s2W1, s2b1, s2W2, s2b2, s3W1, s3b1, s3W2, s3b2,
     s4W1, s4b1, s4W2, s4b2, s5W1, s5b1, s5W2, s5b2,
     wsiW, wsib, Wq, bq, Wk, bv_, Wv, bv, Wo, bo,
     agaW, agab, agbW, agbb, agcW, agcb,
     rhoW, rhob, clsW, clsb,
     out_ref, H_ref, S_ref, Q_ref) = refs
    i = pl.program_id(0)

    @pl.when(i == 0)
    def _init():
        def mlp(x, W1, b1, W2, b2):
            h = x[...] @ W1[...] + b1[...]
            h = jnp.where(h > 0, h, jnp.exp(jnp.minimum(h, 0.0)) - 1.0)
            h = h @ W2[...] + b2[...]
            return jnp.where(h > 0, h, jnp.exp(jnp.minimum(h, 0.0)) - 1.0)

        h_omic = jnp.concatenate([
            mlp(xo1, s0W1, s0b1, s0W2, s0b2),
            mlp(xo2, s1W1, s1b1, s1W2, s1b2),
            mlp(xo3, s2W1, s2b1, s2W2, s2b2),
            mlp(xo4, s3W1, s3b1, s3W2, s3b2),
            mlp(xo5, s4W1, s4b1, s4W2, s4b2),
            mlp(xo6, s5W1, s5b1, s5W2, s5b2),
        ], axis=0)
        q = h_omic @ Wq[...] + bq[...]
        scale = _DH ** -0.5
        zpad = jnp.zeros((2, _HID), jnp.float32)
        parts = []
        for h in range(_NH):
            qs = q[:, h * _DH:(h + 1) * _DH] * scale
            qe = jax.lax.dot_general(
                qs, Wk[...][:, h * _DH:(h + 1) * _DH],
                (((1,), (1,)), ((), ())),
                preferred_element_type=jnp.float32)
            parts.append(qe)
            parts.append(zpad)
        Q_ref[...] = jnp.concatenate(parts, axis=0)

    h_blk = jnp.maximum(
        jnp.dot(x_ref[...].astype(jnp.bfloat16), wsiW[...].astype(jnp.bfloat16),
                preferred_element_type=jnp.float32)
        + wsib[...], 0.0)
    H_ref[pl.ds(i * _BLK, _BLK), :] = h_blk
    S_ref[pl.ds(i * _BLK, _BLK), :] = jax.lax.dot_general(
        h_blk, Q_ref[...], (((1,), (1,)), ((), ())),
        preferred_element_type=jnp.float32)

    @pl.when(i == _NBLK - 1)
    def _final():
        S = S_ref[...]
        m = jnp.max(S, axis=0, keepdims=True)
        e = jnp.exp(S - m)
        a = e / jnp.sum(e, axis=0, keepdims=True)
        ctx = jax.lax.dot_general(
            a, H_ref[...], (((0,), (0,)), ((), ())),
            preferred_element_type=jnp.float32)
        o_parts = []
        for h in range(_NH):
            o_parts.append(
                ctx[h * 8:(h + 1) * 8, :] @ Wv[...][:, h * _DH:(h + 1) * _DH]
                + bv[...][:, h * _DH:(h + 1) * _DH])
        o = jnp.concatenate(o_parts, axis=1)
        hp = o @ Wo[...] + bo[...]
        ga = jnp.tanh(hp @ agaW[...] + agab[...])
        gb = jax.nn.sigmoid(hp @ agbW[...] + agbb[...])
        A = (ga * gb) @ agcW[...] + agcb[...]
        row = jax.lax.broadcasted_iota(jnp.int32, (8, 1), 0)
        A = jnp.where(row < 6, A, -jnp.inf)
        Am = jnp.max(A, axis=0, keepdims=True)
        Ae = jnp.exp(A - Am)
        w = Ae / jnp.sum(Ae, axis=0, keepdims=True)
        hpath = jax.lax.dot_general(
            w, hp, (((0,), (0,)), ((), ())),
            preferred_element_type=jnp.float32)
        hpath = jnp.maximum(hpath @ rhoW[...] + rhob[...], 0.0)
        out_ref[...] = hpath @ clsW[...] + clsb[...]


def kernel(x_path, x_omic1, x_omic2, x_omic3, x_omic4, x_omic5, x_omic6,
           sig0_W1, sig0_b1, sig0_W2, sig0_b2, sig1_W1, sig1_b1, sig1_W2,
           sig1_b2, sig2_W1, sig2_b1, sig2_W2, sig2_b2, sig3_W1, sig3_b1,
           sig3_W2, sig3_b2, sig4_W1, sig4_b1, sig4_W2, sig4_b2, sig5_W1,
           sig5_b1, sig5_W2, sig5_b2, wsi_W, wsi_b, head_W, head_b, tail_W,
           tail_b, l1_W, l1_b, l2_W, l2_b, att1_W, att1_b, att2_W, att2_b,
           mha_Wq, mha_bq, mha_Wk, mha_bk, mha_Wv, mha_bv, mha_Wo, mha_bo,
           ag_a_W, ag_a_b, ag_b_W, ag_b_b, ag_c_W, ag_c_b, rho_W, rho_b,
           cls_W, cls_b):
    r2 = lambda v: v.reshape(1, -1)
    operands = [
        x_path,
        r2(x_omic1), r2(x_omic2), r2(x_omic3),
        r2(x_omic4), r2(x_omic5), r2(x_omic6),
        sig0_W1, r2(sig0_b1), sig0_W2, r2(sig0_b2),
        sig1_W1, r2(sig1_b1), sig1_W2, r2(sig1_b2),
        sig2_W1, r2(sig2_b1), sig2_W2, r2(sig2_b2),
        sig3_W1, r2(sig3_b1), sig3_W2, r2(sig3_b2),
        sig4_W1, r2(sig4_b1), sig4_W2, r2(sig4_b2),
        sig5_W1, r2(sig5_b1), sig5_W2, r2(sig5_b2),
        wsi_W, r2(wsi_b),
        mha_Wq, r2(mha_bq), mha_Wk, r2(mha_bk), mha_Wv, r2(mha_bv),
        mha_Wo, r2(mha_bo),
        ag_a_W, r2(ag_a_b), ag_b_W, r2(ag_b_b), ag_c_W, r2(ag_c_b),
        rho_W, r2(rho_b), cls_W, r2(cls_b),
    ]

    def _rep(shape):
        return pl.BlockSpec(shape, lambda i: (0,) * len(shape))

    in_specs = [pl.BlockSpec((_BLK, 1024), lambda i: (i, 0))]
    in_specs += [_rep(op.shape) for op in operands[1:]]

    return pl.pallas_call(
        _fused,
        grid=(_NBLK,),
        in_specs=in_specs,
        out_specs=pl.BlockSpec((1, 4), lambda i: (0, 0)),
        out_shape=jax.ShapeDtypeStruct((1, 4), jnp.float32),
        scratch_shapes=[
            pltpu.VMEM((_N, _HID), jnp.float32),
            pltpu.VMEM((_N, _QR), jnp.float32),
            pltpu.VMEM((_QR, _HID), jnp.float32),
        ],
    )(*operands)


# speedup vs baseline: 1.1239x; 1.1239x over previous
import jax
import jax.numpy as jnp
from jax.experimental import pallas as pl
from jax.experimental.pallas import tpu as pltpu

_N = 4096
_HID = 256
_NH = 8
_DH = 32
_BLK = 512
_NBLK = _N // _BLK
_QR = _NH * 8


def _fused(*refs):
    (x_ref,
     xo1, xo2, xo3, xo4, xo5, xo6,
     s0W1, s0b1, s0W2, s0b2, s1W1, s1b1, s1W2, s1b2,
     s2W1, s2b1, s2W2, s2b2, s3W1, s3b1, s3W2, s3b2,
     s4W1, s4b1, s4W2, s4b2, s5W1, s5b1, s5W2, s5b2,
     wsiW, wsib, Wq, bq, Wk, bv_, Wv, bv, Wo, bo,
     agaW, agab, agbW, agbb, agcW, agcb,
     rhoW, rhob, clsW, clsb,
     out_ref, H_ref, S_ref, Q_ref) = refs
    i = pl.program_id(0)

    @pl.when(i == 0)
    def _init():
        def mlp(x, W1, b1, W2, b2):
            h = x[...][None, :] @ W1[...] + b1[...]
            h = jnp.where(h > 0, h, jnp.exp(jnp.minimum(h, 0.0)) - 1.0)
            h = h @ W2[...] + b2[...]
            return jnp.where(h > 0, h, jnp.exp(jnp.minimum(h, 0.0)) - 1.0)

        h_omic = jnp.concatenate([
            mlp(xo1, s0W1, s0b1, s0W2, s0b2),
            mlp(xo2, s1W1, s1b1, s1W2, s1b2),
            mlp(xo3, s2W1, s2b1, s2W2, s2b2),
            mlp(xo4, s3W1, s3b1, s3W2, s3b2),
            mlp(xo5, s4W1, s4b1, s4W2, s4b2),
            mlp(xo6, s5W1, s5b1, s5W2, s5b2),
        ], axis=0)
        q = h_omic @ Wq[...] + bq[...]
        scale = _DH ** -0.5
        zpad = jnp.zeros((2, _HID), jnp.float32)
        parts = []
        for h in range(_NH):
            qs = q[:, h * _DH:(h + 1) * _DH] * scale
            qe = jax.lax.dot_general(
                qs, Wk[...][:, h * _DH:(h + 1) * _DH],
                (((1,), (1,)), ((), ())),
                preferred_element_type=jnp.float32)
            parts.append(qe)
            parts.append(zpad)
        Q_ref[...] = jnp.concatenate(parts, axis=0)

    h_blk = jnp.maximum(
        jnp.dot(x_ref[...].astype(jnp.bfloat16), wsiW[...].astype(jnp.bfloat16),
                preferred_element_type=jnp.float32)
        + wsib[...], 0.0)
    H_ref[pl.ds(i * _BLK, _BLK), :] = h_blk
    S_ref[pl.ds(i * _BLK, _BLK), :] = jax.lax.dot_general(
        h_blk, Q_ref[...], (((1,), (1,)), ((), ())),
        preferred_element_type=jnp.float32)

    @pl.when(i == _NBLK - 1)
    def _final():
        S = S_ref[...]
        m = jnp.max(S, axis=0, keepdims=True)
        e = jnp.exp(S - m)
        a = e / jnp.sum(e, axis=0, keepdims=True)
        ctx = jax.lax.dot_general(
            a, H_ref[...], (((0,), (0,)), ((), ())),
            preferred_element_type=jnp.float32)
        o_parts = []
        for h in range(_NH):
            o_parts.append(
                ctx[h * 8:(h + 1) * 8, :] @ Wv[...][:, h * _DH:(h + 1) * _DH]
                + bv[...][h * _DH:(h + 1) * _DH])
        o = jnp.concatenate(o_parts, axis=1)
        hp = o @ Wo[...] + bo[...]
        ga = jnp.tanh(hp @ agaW[...] + agab[...])
        gb = jax.nn.sigmoid(hp @ agbW[...] + agbb[...])
        A = (ga * gb) @ agcW[...] + agcb[...]
        row = jax.lax.broadcasted_iota(jnp.int32, (8, 1), 0)
        A = jnp.where(row < 6, A, -jnp.inf)
        Am = jnp.max(A, axis=0, keepdims=True)
        Ae = jnp.exp(A - Am)
        w = Ae / jnp.sum(Ae, axis=0, keepdims=True)
        hpath = jax.lax.dot_general(
            w, hp, (((0,), (0,)), ((), ())),
            preferred_element_type=jnp.float32)
        hpath = jnp.maximum(hpath @ rhoW[...] + rhob[...], 0.0)
        out_ref[...] = hpath @ clsW[...] + clsb[...]


def kernel(x_path, x_omic1, x_omic2, x_omic3, x_omic4, x_omic5, x_omic6,
           sig0_W1, sig0_b1, sig0_W2, sig0_b2, sig1_W1, sig1_b1, sig1_W2,
           sig1_b2, sig2_W1, sig2_b1, sig2_W2, sig2_b2, sig3_W1, sig3_b1,
           sig3_W2, sig3_b2, sig4_W1, sig4_b1, sig4_W2, sig4_b2, sig5_W1,
           sig5_b1, sig5_W2, sig5_b2, wsi_W, wsi_b, head_W, head_b, tail_W,
           tail_b, l1_W, l1_b, l2_W, l2_b, att1_W, att1_b, att2_W, att2_b,
           mha_Wq, mha_bq, mha_Wk, mha_bk, mha_Wv, mha_bv, mha_Wo, mha_bo,
           ag_a_W, ag_a_b, ag_b_W, ag_b_b, ag_c_W, ag_c_b, rho_W, rho_b,
           cls_W, cls_b):
    operands = [
        x_path,
        x_omic1, x_omic2, x_omic3, x_omic4, x_omic5, x_omic6,
        sig0_W1, sig0_b1, sig0_W2, sig0_b2,
        sig1_W1, sig1_b1, sig1_W2, sig1_b2,
        sig2_W1, sig2_b1, sig2_W2, sig2_b2,
        sig3_W1, sig3_b1, sig3_W2, sig3_b2,
        sig4_W1, sig4_b1, sig4_W2, sig4_b2,
        sig5_W1, sig5_b1, sig5_W2, sig5_b2,
        wsi_W, wsi_b,
        mha_Wq, mha_bq, mha_Wk, mha_bk, mha_Wv, mha_bv,
        mha_Wo, mha_bo,
        ag_a_W, ag_a_b, ag_b_W, ag_b_b, ag_c_W, ag_c_b,
        rho_W, rho_b, cls_W, cls_b,
    ]

    def _rep(shape):
        return pl.BlockSpec(shape, lambda i: (0,) * len(shape))

    in_specs = [pl.BlockSpec((_BLK, 1024), lambda i: (i, 0))]
    in_specs += [_rep(op.shape) for op in operands[1:]]

    return pl.pallas_call(
        _fused,
        grid=(_NBLK,),
        in_specs=in_specs,
        out_specs=pl.BlockSpec((1, 4), lambda i: (0, 0)),
        out_shape=jax.ShapeDtypeStruct((1, 4), jnp.float32),
        scratch_shapes=[
            pltpu.VMEM((_N, _HID), jnp.float32),
            pltpu.VMEM((_N, _QR), jnp.float32),
            pltpu.VMEM((_QR, _HID), jnp.float32),
        ],
    )(*operands)


# speedup vs baseline: 1.1376x; 1.0122x over previous
import jax
import jax.numpy as jnp
from jax.experimental import pallas as pl
from jax.experimental.pallas import tpu as pltpu

_N = 4096
_HID = 256
_NH = 8
_DH = 32
_BLK = 512
_NBLK = _N // _BLK
_QR = _NH * 8

_TAIL_NAMES = (
    'x_omic1', 'x_omic2', 'x_omic3', 'x_omic4', 'x_omic5', 'x_omic6',
    'sig0_W1', 'sig0_b1', 'sig0_W2', 'sig0_b2',
    'sig1_W1', 'sig1_b1', 'sig1_W2', 'sig1_b2',
    'sig2_W1', 'sig2_b1', 'sig2_W2', 'sig2_b2',
    'sig3_W1', 'sig3_b1', 'sig3_W2', 'sig3_b2',
    'sig4_W1', 'sig4_b1', 'sig4_W2', 'sig4_b2',
    'sig5_W1', 'sig5_b1', 'sig5_W2', 'sig5_b2',
    'mha_Wq', 'mha_bq', 'mha_Wk', 'mha_Wv', 'mha_bv', 'mha_Wo', 'mha_bo',
    'ag_a_W', 'ag_a_b', 'ag_b_W', 'ag_b_b', 'ag_c_W', 'ag_c_b',
    'rho_W', 'rho_b', 'cls_W', 'cls_b',
)
_NT = len(_TAIL_NAMES)


def _fused(*refs):
    x_ref, wsiW, wsib = refs[0], refs[1], refs[2]
    hbm = refs[3:3 + _NT]
    out_ref = refs[3 + _NT]
    H_ref = refs[4 + _NT]
    vmem = refs[5 + _NT:5 + 2 * _NT]
    sem = refs[5 + 2 * _NT]
    i = pl.program_id(0)

    @pl.when(i == 0)
    def _start_dmas():
        for j in range(_NT):
            pltpu.make_async_copy(hbm[j], vmem[j], sem.at[j]).start()

    h_blk = jnp.maximum(
        jnp.dot(x_ref[...].astype(jnp.bfloat16), wsiW[...].astype(jnp.bfloat16),
                preferred_element_type=jnp.float32)
        + wsib[...], 0.0)
    H_ref[pl.ds(i * _BLK, _BLK), :] = h_blk

    @pl.when(i == _NBLK - 1)
    def _final():
        for j in range(_NT):
            pltpu.make_async_copy(hbm[j], vmem[j], sem.at[j]).wait()
        w = dict(zip(_TAIL_NAMES, vmem))

        def mlp(x, W1, b1, W2, b2):
            h = x[...][None, :] @ W1[...] + b1[...]
            h = jnp.where(h > 0, h, jnp.exp(jnp.minimum(h, 0.0)) - 1.0)
            h = h @ W2[...] + b2[...]
            return jnp.where(h > 0, h, jnp.exp(jnp.minimum(h, 0.0)) - 1.0)

        h_omic = jnp.concatenate([
            mlp(w['x_omic%d' % (k + 1)], w['sig%d_W1' % k], w['sig%d_b1' % k],
                w['sig%d_W2' % k], w['sig%d_b2' % k])
            for k in range(6)
        ], axis=0)
        q = h_omic @ w['mha_Wq'][...] + w['mha_bq'][...]
        scale = _DH ** -0.5
        zpad = jnp.zeros((2, _HID), jnp.float32)
        parts = []
        for h in range(_NH):
            qs = q[:, h * _DH:(h + 1) * _DH] * scale
            qe = jax.lax.dot_general(
                qs, w['mha_Wk'][...][:, h * _DH:(h + 1) * _DH],
                (((1,), (1,)), ((), ())),
                preferred_element_type=jnp.float32)
            parts.append(qe)
            parts.append(zpad)
        Q = jnp.concatenate(parts, axis=0)

        S = jax.lax.dot_general(
            H_ref[...], Q, (((1,), (1,)), ((), ())),
            preferred_element_type=jnp.float32)
        m = jnp.max(S, axis=0, keepdims=True)
        e = jnp.exp(S - m)
        a = e / jnp.sum(e, axis=0, keepdims=True)
        ctx = jax.lax.dot_general(
            a, H_ref[...], (((0,), (0,)), ((), ())),
            preferred_element_type=jnp.float32)
        o_parts = []
        for h in range(_NH):
            o_parts.append(
                ctx[h * 8:(h + 1) * 8, :]
                @ w['mha_Wv'][...][:, h * _DH:(h + 1) * _DH]
                + w['mha_bv'][...][h * _DH:(h + 1) * _DH])
        o = jnp.concatenate(o_parts, axis=1)
        hp = o @ w['mha_Wo'][...] + w['mha_bo'][...]
        ga = jnp.tanh(hp @ w['ag_a_W'][...] + w['ag_a_b'][...])
        gb = jax.nn.sigmoid(hp @ w['ag_b_W'][...] + w['ag_b_b'][...])
        A = (ga * gb) @ w['ag_c_W'][...] + w['ag_c_b'][...]
        row = jax.lax.broadcasted_iota(jnp.int32, (8, 1), 0)
        A = jnp.where(row < 6, A, -jnp.inf)
        Am = jnp.max(A, axis=0, keepdims=True)
        Ae = jnp.exp(A - Am)
        wp = Ae / jnp.sum(Ae, axis=0, keepdims=True)
        hpath = jax.lax.dot_general(
            wp, hp, (((0,), (0,)), ((), ())),
            preferred_element_type=jnp.float32)
        hpath = jnp.maximum(hpath @ w['rho_W'][...] + w['rho_b'][...], 0.0)
        out_ref[...] = hpath @ w['cls_W'][...] + w['cls_b'][...]


def kernel(x_path, x_omic1, x_omic2, x_omic3, x_omic4, x_omic5, x_omic6,
           sig0_W1, sig0_b1, sig0_W2, sig0_b2, sig1_W1, sig1_b1, sig1_W2,
           sig1_b2, sig2_W1, sig2_b1, sig2_W2, sig2_b2, sig3_W1, sig3_b1,
           sig3_W2, sig3_b2, sig4_W1, sig4_b1, sig4_W2, sig4_b2, sig5_W1,
           sig5_b1, sig5_W2, sig5_b2, wsi_W, wsi_b, head_W, head_b, tail_W,
           tail_b, l1_W, l1_b, l2_W, l2_b, att1_W, att1_b, att2_W, att2_b,
           mha_Wq, mha_bq, mha_Wk, mha_bk, mha_Wv, mha_bv, mha_Wo, mha_bo,
           ag_a_W, ag_a_b, ag_b_W, ag_b_b, ag_c_W, ag_c_b, rho_W, rho_b,
           cls_W, cls_b):
    scope = locals()
    tail_ops = [scope[nm] for nm in _TAIL_NAMES]
    operands = [x_path, wsi_W, wsi_b] + tail_ops

    def _rep(shape):
        return pl.BlockSpec(shape, lambda i: (0,) * len(shape))

    in_specs = [
        pl.BlockSpec((_BLK, 1024), lambda i: (i, 0)),
        _rep(wsi_W.shape),
        _rep(wsi_b.shape),
    ] + [pl.BlockSpec(memory_space=pltpu.MemorySpace.HBM)
         for _ in range(_NT)]

    return pl.pallas_call(
        _fused,
        grid=(_NBLK,),
        in_specs=in_specs,
        out_specs=pl.BlockSpec((1, 4), lambda i: (0, 0)),
        out_shape=jax.ShapeDtypeStruct((1, 4), jnp.float32),
        scratch_shapes=(
            [pltpu.VMEM((_N, _HID), jnp.float32)]
            + [pltpu.VMEM(op.shape, op.dtype) for op in tail_ops]
            + [pltpu.SemaphoreType.DMA((_NT,))]
        ),
    )(*operands)


# speedup vs baseline: 1.2862x; 1.1306x over previous
import jax
import jax.numpy as jnp
from jax.experimental import pallas as pl
from jax.experimental.pallas import tpu as pltpu

_N = 4096
_HID = 256
_NH = 8
_DH = 32
_BLK = 1024
_NBLK = _N // _BLK
_QR = _NH * 8

_TAIL_NAMES = (
    'x_omic1', 'x_omic2', 'x_omic3', 'x_omic4', 'x_omic5', 'x_omic6',
    'sig0_W1', 'sig0_b1', 'sig0_W2', 'sig0_b2',
    'sig1_W1', 'sig1_b1', 'sig1_W2', 'sig1_b2',
    'sig2_W1', 'sig2_b1', 'sig2_W2', 'sig2_b2',
    'sig3_W1', 'sig3_b1', 'sig3_W2', 'sig3_b2',
    'sig4_W1', 'sig4_b1', 'sig4_W2', 'sig4_b2',
    'sig5_W1', 'sig5_b1', 'sig5_W2', 'sig5_b2',
    'mha_Wq', 'mha_bq', 'mha_Wk', 'mha_Wv', 'mha_bv', 'mha_Wo', 'mha_bo',
    'ag_a_W', 'ag_a_b', 'ag_b_W', 'ag_b_b', 'ag_c_W', 'ag_c_b',
    'rho_W', 'rho_b', 'cls_W', 'cls_b',
)
_NT = len(_TAIL_NAMES)


def _fused(*refs):
    x_ref, wsiW, wsib = refs[0], refs[1], refs[2]
    hbm = refs[3:3 + _NT]
    out_ref = refs[3 + _NT]
    H_ref = refs[4 + _NT]
    vmem = refs[5 + _NT:5 + 2 * _NT]
    sem = refs[5 + 2 * _NT]
    i = pl.program_id(0)

    @pl.when(i == 0)
    def _start_dmas():
        for j in range(_NT):
            pltpu.make_async_copy(hbm[j], vmem[j], sem.at[j]).start()

    h_blk = jnp.maximum(
        jnp.dot(x_ref[...].astype(jnp.bfloat16), wsiW[...].astype(jnp.bfloat16),
                preferred_element_type=jnp.float32)
        + wsib[...], 0.0)
    H_ref[pl.ds(i * _BLK, _BLK), :] = h_blk

    @pl.when(i == _NBLK - 1)
    def _final():
        for j in range(_NT):
            pltpu.make_async_copy(hbm[j], vmem[j], sem.at[j]).wait()
        w = dict(zip(_TAIL_NAMES, vmem))

        def mlp(x, W1, b1, W2, b2):
            h = x[...][None, :] @ W1[...] + b1[...]
            h = jnp.where(h > 0, h, jnp.exp(jnp.minimum(h, 0.0)) - 1.0)
            h = h @ W2[...] + b2[...]
            return jnp.where(h > 0, h, jnp.exp(jnp.minimum(h, 0.0)) - 1.0)

        h_omic = jnp.concatenate([
            mlp(w['x_omic%d' % (k + 1)], w['sig%d_W1' % k], w['sig%d_b1' % k],
                w['sig%d_W2' % k], w['sig%d_b2' % k])
            for k in range(6)
        ], axis=0)
        q = h_omic @ w['mha_Wq'][...] + w['mha_bq'][...]
        scale = _DH ** -0.5
        zpad = jnp.zeros((2, _HID), jnp.float32)
        parts = []
        for h in range(_NH):
            qs = q[:, h * _DH:(h + 1) * _DH] * scale
            qe = jax.lax.dot_general(
                qs, w['mha_Wk'][...][:, h * _DH:(h + 1) * _DH],
                (((1,), (1,)), ((), ())),
                preferred_element_type=jnp.float32)
            parts.append(qe)
            parts.append(zpad)
        Q = jnp.concatenate(parts, axis=0)

        S = jax.lax.dot_general(
            H_ref[...], Q, (((1,), (1,)), ((), ())),
            preferred_element_type=jnp.float32)
        m = jnp.max(S, axis=0, keepdims=True)
        e = jnp.exp(S - m)
        a = e / jnp.sum(e, axis=0, keepdims=True)
        ctx = jax.lax.dot_general(
            a, H_ref[...], (((0,), (0,)), ((), ())),
            preferred_element_type=jnp.float32)
        o_parts = []
        for h in range(_NH):
            o_parts.append(
                ctx[h * 8:(h + 1) * 8, :]
                @ w['mha_Wv'][...][:, h * _DH:(h + 1) * _DH]
                + w['mha_bv'][...][h * _DH:(h + 1) * _DH])
        o = jnp.concatenate(o_parts, axis=1)
        hp = o @ w['mha_Wo'][...] + w['mha_bo'][...]
        ga = jnp.tanh(hp @ w['ag_a_W'][...] + w['ag_a_b'][...])
        gb = jax.nn.sigmoid(hp @ w['ag_b_W'][...] + w['ag_b_b'][...])
        A = (ga * gb) @ w['ag_c_W'][...] + w['ag_c_b'][...]
        row = jax.lax.broadcasted_iota(jnp.int32, (8, 1), 0)
        A = jnp.where(row < 6, A, -jnp.inf)
        Am = jnp.max(A, axis=0, keepdims=True)
        Ae = jnp.exp(A - Am)
        wp = Ae / jnp.sum(Ae, axis=0, keepdims=True)
        hpath = jax.lax.dot_general(
            wp, hp, (((0,), (0,)), ((), ())),
            preferred_element_type=jnp.float32)
        hpath = jnp.maximum(hpath @ w['rho_W'][...] + w['rho_b'][...], 0.0)
        out_ref[...] = hpath @ w['cls_W'][...] + w['cls_b'][...]


def kernel(x_path, x_omic1, x_omic2, x_omic3, x_omic4, x_omic5, x_omic6,
           sig0_W1, sig0_b1, sig0_W2, sig0_b2, sig1_W1, sig1_b1, sig1_W2,
           sig1_b2, sig2_W1, sig2_b1, sig2_W2, sig2_b2, sig3_W1, sig3_b1,
           sig3_W2, sig3_b2, sig4_W1, sig4_b1, sig4_W2, sig4_b2, sig5_W1,
           sig5_b1, sig5_W2, sig5_b2, wsi_W, wsi_b, head_W, head_b, tail_W,
           tail_b, l1_W, l1_b, l2_W, l2_b, att1_W, att1_b, att2_W, att2_b,
           mha_Wq, mha_bq, mha_Wk, mha_bk, mha_Wv, mha_bv, mha_Wo, mha_bo,
           ag_a_W, ag_a_b, ag_b_W, ag_b_b, ag_c_W, ag_c_b, rho_W, rho_b,
           cls_W, cls_b):
    scope = locals()
    tail_ops = [scope[nm] for nm in _TAIL_NAMES]
    operands = [x_path, wsi_W, wsi_b] + tail_ops

    def _rep(shape):
        return pl.BlockSpec(shape, lambda i: (0,) * len(shape))

    in_specs = [
        pl.BlockSpec((_BLK, 1024), lambda i: (i, 0)),
        _rep(wsi_W.shape),
        _rep(wsi_b.shape),
    ] + [pl.BlockSpec(memory_space=pltpu.MemorySpace.HBM)
         for _ in range(_NT)]

    return pl.pallas_call(
        _fused,
        grid=(_NBLK,),
        in_specs=in_specs,
        out_specs=pl.BlockSpec((1, 4), lambda i: (0, 0)),
        out_shape=jax.ShapeDtypeStruct((1, 4), jnp.float32),
        scratch_shapes=(
            [pltpu.VMEM((_N, _HID), jnp.float32)]
            + [pltpu.VMEM(op.shape, op.dtype) for op in tail_ops]
            + [pltpu.SemaphoreType.DMA((_NT,))]
        ),
    )(*operands)


# speedup vs baseline: 1.3594x; 1.0569x over previous
import jax
import jax.numpy as jnp
from jax.experimental import pallas as pl
from jax.experimental.pallas import tpu as pltpu

_N = 4096
_HID = 256
_NH = 8
_DH = 32
_BLK = 2048
_NBLK = _N // _BLK
_QR = _NH * 8

_TAIL_NAMES = (
    'x_omic1', 'x_omic2', 'x_omic3', 'x_omic4', 'x_omic5', 'x_omic6',
    'sig0_W1', 'sig0_b1', 'sig0_W2', 'sig0_b2',
    'sig1_W1', 'sig1_b1', 'sig1_W2', 'sig1_b2',
    'sig2_W1', 'sig2_b1', 'sig2_W2', 'sig2_b2',
    'sig3_W1', 'sig3_b1', 'sig3_W2', 'sig3_b2',
    'sig4_W1', 'sig4_b1', 'sig4_W2', 'sig4_b2',
    'sig5_W1', 'sig5_b1', 'sig5_W2', 'sig5_b2',
    'mha_Wq', 'mha_bq', 'mha_Wk', 'mha_Wv', 'mha_bv', 'mha_Wo', 'mha_bo',
    'ag_a_W', 'ag_a_b', 'ag_b_W', 'ag_b_b', 'ag_c_W', 'ag_c_b',
    'rho_W', 'rho_b', 'cls_W', 'cls_b',
)
_NT = len(_TAIL_NAMES)


def _fused(*refs):
    x_ref, wsiW, wsib = refs[0], refs[1], refs[2]
    hbm = refs[3:3 + _NT]
    out_ref = refs[3 + _NT]
    H_ref = refs[4 + _NT]
    vmem = refs[5 + _NT:5 + 2 * _NT]
    sem = refs[5 + 2 * _NT]
    i = pl.program_id(0)

    @pl.when(i == 0)
    def _start_dmas():
        for j in range(_NT):
            pltpu.make_async_copy(hbm[j], vmem[j], sem.at[j]).start()

    h_blk = jnp.maximum(
        jnp.dot(x_ref[...].astype(jnp.bfloat16), wsiW[...].astype(jnp.bfloat16),
                preferred_element_type=jnp.float32)
        + wsib[...], 0.0)
    H_ref[pl.ds(i * _BLK, _BLK), :] = h_blk

    @pl.when(i == _NBLK - 1)
    def _final():
        for j in range(_NT):
            pltpu.make_async_copy(hbm[j], vmem[j], sem.at[j]).wait()
        w = dict(zip(_TAIL_NAMES, vmem))

        def mlp(x, W1, b1, W2, b2):
            h = x[...][None, :] @ W1[...] + b1[...]
            h = jnp.where(h > 0, h, jnp.exp(jnp.minimum(h, 0.0)) - 1.0)
            h = h @ W2[...] + b2[...]
            return jnp.where(h > 0, h, jnp.exp(jnp.minimum(h, 0.0)) - 1.0)

        h_omic = jnp.concatenate([
            mlp(w['x_omic%d' % (k + 1)], w['sig%d_W1' % k], w['sig%d_b1' % k],
                w['sig%d_W2' % k], w['sig%d_b2' % k])
            for k in range(6)
        ], axis=0)
        q = h_omic @ w['mha_Wq'][...] + w['mha_bq'][...]
        scale = _DH ** -0.5
        zpad = jnp.zeros((2, _HID), jnp.float32)
        parts = []
        for h in range(_NH):
            qs = q[:, h * _DH:(h + 1) * _DH] * scale
            qe = jax.lax.dot_general(
                qs, w['mha_Wk'][...][:, h * _DH:(h + 1) * _DH],
                (((1,), (1,)), ((), ())),
                preferred_element_type=jnp.float32)
            parts.append(qe)
            parts.append(zpad)
        Q = jnp.concatenate(parts, axis=0)

        S = jax.lax.dot_general(
            H_ref[...], Q, (((1,), (1,)), ((), ())),
            preferred_element_type=jnp.float32)
        m = jnp.max(S, axis=0, keepdims=True)
        e = jnp.exp(S - m)
        a = e / jnp.sum(e, axis=0, keepdims=True)
        ctx = jax.lax.dot_general(
            a, H_ref[...], (((0,), (0,)), ((), ())),
            preferred_element_type=jnp.float32)
        o_parts = []
        for h in range(_NH):
            o_parts.append(
                ctx[h * 8:(h + 1) * 8, :]
                @ w['mha_Wv'][...][:, h * _DH:(h + 1) * _DH]
                + w['mha_bv'][...][h * _DH:(h + 1) * _DH])
        o = jnp.concatenate(o_parts, axis=1)
        hp = o @ w['mha_Wo'][...] + w['mha_bo'][...]
        ga = jnp.tanh(hp @ w['ag_a_W'][...] + w['ag_a_b'][...])
        gb = jax.nn.sigmoid(hp @ w['ag_b_W'][...] + w['ag_b_b'][...])
        A = (ga * gb) @ w['ag_c_W'][...] + w['ag_c_b'][...]
        row = jax.lax.broadcasted_iota(jnp.int32, (8, 1), 0)
        A = jnp.where(row < 6, A, -jnp.inf)
        Am = jnp.max(A, axis=0, keepdims=True)
        Ae = jnp.exp(A - Am)
        wp = Ae / jnp.sum(Ae, axis=0, keepdims=True)
        hpath = jax.lax.dot_general(
            wp, hp, (((0,), (0,)), ((), ())),
            preferred_element_type=jnp.float32)
        hpath = jnp.maximum(hpath @ w['rho_W'][...] + w['rho_b'][...], 0.0)
        out_ref[...] = hpath @ w['cls_W'][...] + w['cls_b'][...]


def kernel(x_path, x_omic1, x_omic2, x_omic3, x_omic4, x_omic5, x_omic6,
           sig0_W1, sig0_b1, sig0_W2, sig0_b2, sig1_W1, sig1_b1, sig1_W2,
           sig1_b2, sig2_W1, sig2_b1, sig2_W2, sig2_b2, sig3_W1, sig3_b1,
           sig3_W2, sig3_b2, sig4_W1, sig4_b1, sig4_W2, sig4_b2, sig5_W1,
           sig5_b1, sig5_W2, sig5_b2, wsi_W, wsi_b, head_W, head_b, tail_W,
           tail_b, l1_W, l1_b, l2_W, l2_b, att1_W, att1_b, att2_W, att2_b,
           mha_Wq, mha_bq, mha_Wk, mha_bk, mha_Wv, mha_bv, mha_Wo, mha_bo,
           ag_a_W, ag_a_b, ag_b_W, ag_b_b, ag_c_W, ag_c_b, rho_W, rho_b,
           cls_W, cls_b):
    scope = locals()
    tail_ops = [scope[nm] for nm in _TAIL_NAMES]
    operands = [x_path, wsi_W, wsi_b] + tail_ops

    def _rep(shape):
        return pl.BlockSpec(shape, lambda i: (0,) * len(shape))

    in_specs = [
        pl.BlockSpec((_BLK, 1024), lambda i: (i, 0)),
        _rep(wsi_W.shape),
        _rep(wsi_b.shape),
    ] + [pl.BlockSpec(memory_space=pltpu.MemorySpace.HBM)
         for _ in range(_NT)]

    return pl.pallas_call(
        _fused,
        grid=(_NBLK,),
        in_specs=in_specs,
        out_specs=pl.BlockSpec((1, 4), lambda i: (0, 0)),
        out_shape=jax.ShapeDtypeStruct((1, 4), jnp.float32),
        scratch_shapes=(
            [pltpu.VMEM((_N, _HID), jnp.float32)]
            + [pltpu.VMEM(op.shape, op.dtype) for op in tail_ops]
            + [pltpu.SemaphoreType.DMA((_NT,))]
        ),
    )(*operands)
